# baseline (speedup 1.0000x reference)
import functools

import numpy as np
import jax
from jax import lax
import jax.numpy as jnp
from jax.experimental import pallas as pl
from jax.experimental.pallas import tpu as pltpu
from jax.experimental.pallas import tpu_sc as plsc

_DIM = 64
_PAD = 128
_SEQ = 200
_BATCH = 4096
_LANES = 16
_NW = 32
_BB = _BATCH // _PAD
_ITEMS_PW = _SEQ * _BB // _NW
_RING = 4


def _positional_encoding(length, dim):
    depth = dim / 2
    positions = np.arange(length)[:, np.newaxis]
    depths = np.arange(int(depth))[np.newaxis, :] / depth
    angle_rates = 1 / 10000 ** depths
    angle_rads = positions * angle_rates
    return np.concatenate(
        [np.sin(angle_rads), np.cos(angle_rads)], axis=-1
    ).astype(np.float32)


_POS = _positional_encoding(_SEQ, _DIM).reshape(-1)


@jax.jit
def _embed_sc(xt1, table_p, pos):
    mesh = plsc.VectorSubcoreMesh(
        core_axis_name="core", subcore_axis_name="subcore"
    )

    @functools.partial(
        pl.kernel,
        out_type=jax.ShapeDtypeStruct((_SEQ, _DIM, _BATCH), jnp.float32),
        mesh=mesh,
        scratch_types=[
            pltpu.VMEM((_SEQ * _DIM,), jnp.float32),
            pltpu.VMEM((_ITEMS_PW * _PAD,), jnp.int32),
            [pltpu.VMEM((_PAD, _PAD), jnp.float32) for _ in range(_RING)],
            [pltpu.VMEM((_DIM, _PAD), jnp.float32) for _ in range(2)],
            [pltpu.SemaphoreType.DMA for _ in range(_RING)],
            [pltpu.SemaphoreType.DMA for _ in range(2)],
        ],
        compiler_params=pltpu.CompilerParams(
            use_tc_tiling_on_sc=True, needs_layout_passes=False
        ),
    )
    def k(x_hbm, table_hbm, pos_hbm, o_hbm,
          pos_v, idx_v, rows, tbufs, gsems, osems):
        cid = lax.axis_index("core")
        sid = lax.axis_index("subcore")
        wid = sid * 2 + cid
        it0 = wid * _ITEMS_PW

        pltpu.sync_copy(pos_hbm, pos_v)
        pltpu.sync_copy(x_hbm.at[pl.ds(it0 * _PAD, _ITEMS_PW * _PAD)], idx_v)

        def gather_desc(il, b):
            return pltpu.make_async_copy(
                table_hbm.at[idx_v.at[pl.ds(il * _PAD, _PAD)]],
                rows[b],
                gsems[b],
            )

        def out_desc(il, m):
            it = it0 + il
            t = it // _BB
            b0 = (it % _BB) * _PAD
            return pltpu.make_async_copy(
                tbufs[m],
                o_hbm.at[t].at[pl.ds(0, _DIM), pl.ds(b0, _PAD)],
                osems[m],
            )

        iota = lax.iota(jnp.int32, _LANES)
        idx_rs = [iota + _LANES * kk for kk in range(_PAD // _LANES)]

        def compute(il, b, m):
            it = it0 + il
            t = it // _BB
            src = rows[b]
            dst = tbufs[m]

            @plsc.parallel_loop(0, _DIM // _LANES)
            def _(g):
                pv = pos_v[pl.ds(t * _DIM + g * _LANES, _LANES)]
                for fi in range(_LANES):
                    f_val = g * _LANES + fi
                    p = pv[fi]
                    idx_c = jnp.zeros((_LANES,), jnp.int32) + f_val
                    row_d = dst.at[f_val]
                    for kk in range(_PAD // _LANES):
                        v = plsc.load_gather(src, [idx_rs[kk], idx_c])
                        row_d.at[pl.ds(kk * _LANES, _LANES)][...] = v * 8.0 + p

        gather_desc(0, 0).start()
        gather_desc(1, 1).start()
        gather_desc(2, 2).start()

        @pl.loop(0, _ITEMS_PW, step=_RING)
        def _(j):
            for b in range(_RING):
                il = j + b
                m = b % 2
                gather_desc(il, b).wait()
                if b < 2:
                    @pl.when(j > 0)
                    def _():
                        out_desc(il - 2, m).wait()
                else:
                    out_desc(il - 2, m).wait()
                compute(il, b, m)
                out_desc(il, m).start()

                q3 = j + b + 3

                @pl.when(q3 < _ITEMS_PW)
                def _():
                    gather_desc(il + 3, (b + 3) % _RING).start()

        out_desc(_ITEMS_PW - 2, 0).wait()
        out_desc(_ITEMS_PW - 1, 1).wait()

    return k(xt1, table_p, pos)


def kernel(x, table):
    xt1 = x.astype(jnp.int32).T.reshape(-1)
    table_p = jnp.pad(table, ((0, 0), (0, _PAD - _DIM)))
    out3 = _embed_sc(xt1, table_p, jnp.asarray(_POS))
    return out3.transpose(2, 0, 1)


# speedup vs baseline: 1.3231x; 1.3231x over previous
import functools

import numpy as np
import jax
from jax import lax
import jax.numpy as jnp
from jax.experimental import pallas as pl
from jax.experimental.pallas import tpu as pltpu
from jax.experimental.pallas import tpu_sc as plsc

_DIM = 64
_PAD = 128
_SEQ = 200
_BATCH = 4096
_LANES = 16
_NW = 32
_BB = _BATCH // _PAD
_ITEMS_PW = _SEQ * _BB // _NW
_RING = 4


def _positional_encoding(length, dim):
    depth = dim / 2
    positions = np.arange(length)[:, np.newaxis]
    depths = np.arange(int(depth))[np.newaxis, :] / depth
    angle_rates = 1 / 10000 ** depths
    angle_rads = positions * angle_rates
    return np.concatenate(
        [np.sin(angle_rads), np.cos(angle_rads)], axis=-1
    ).astype(np.float32)


_POS = _positional_encoding(_SEQ, _DIM).reshape(-1)


@jax.jit
def _embed_sc(xt1, table_p, pos):
    mesh = plsc.VectorSubcoreMesh(
        core_axis_name="core", subcore_axis_name="subcore"
    )

    @functools.partial(
        pl.kernel,
        out_type=jax.ShapeDtypeStruct((_SEQ, _DIM, _BATCH), jnp.float32),
        mesh=mesh,
        scratch_types=[
            pltpu.VMEM((_SEQ * _DIM,), jnp.float32),
            pltpu.VMEM((_ITEMS_PW * _PAD,), jnp.int32),
            [pltpu.VMEM((_PAD, _PAD), jnp.float32) for _ in range(_RING)],
            [pltpu.VMEM((_DIM, _PAD), jnp.float32) for _ in range(2)],
            [pltpu.SemaphoreType.DMA for _ in range(_RING)],
            [pltpu.SemaphoreType.DMA for _ in range(2)],
        ],
        compiler_params=pltpu.CompilerParams(
            use_tc_tiling_on_sc=True, needs_layout_passes=False
        ),
    )
    def k(x_hbm, table_hbm, pos_hbm, o_hbm,
          pos_v, idx_v, rows, tbufs, gsems, osems):
        cid = lax.axis_index("core")
        sid = lax.axis_index("subcore")
        wid = sid * 2 + cid
        it0 = wid * _ITEMS_PW

        pltpu.sync_copy(pos_hbm, pos_v)
        pltpu.sync_copy(x_hbm.at[pl.ds(it0 * _PAD, _ITEMS_PW * _PAD)], idx_v)

        def gather_desc(il, b):
            return pltpu.make_async_copy(
                table_hbm.at[idx_v.at[pl.ds(il * _PAD, _PAD)]],
                rows[b],
                gsems[b],
            )

        def out_desc(il, m):
            it = it0 + il
            t = it // _BB
            b0 = (it % _BB) * _PAD
            return pltpu.make_async_copy(
                tbufs[m],
                o_hbm.at[t].at[pl.ds(0, _DIM), pl.ds(b0, _PAD)],
                osems[m],
            )

        iota = lax.iota(jnp.int32, _LANES)
        idx_fs = [iota + _LANES * cc for cc in range(_DIM // _LANES)]

        def compute(il, b, m):
            it = it0 + il
            t = it // _BB
            src = rows[b]
            dst = tbufs[m]
            pos_c = [
                pos_v[pl.ds(t * _DIM + _LANES * cc, _LANES)]
                for cc in range(_DIM // _LANES)
            ]

            @plsc.parallel_loop(0, _PAD, step=1, unroll=4)
            def _(r):
                row_s = src.at[r]
                idx_b = jnp.zeros((_LANES,), jnp.int32) + r
                for cc in range(_DIM // _LANES):
                    v = row_s.at[pl.ds(cc * _LANES, _LANES)][...]
                    plsc.store_scatter(
                        dst, [idx_fs[cc], idx_b], v * 8.0 + pos_c[cc]
                    )

        gather_desc(0, 0).start()
        gather_desc(1, 1).start()
        gather_desc(2, 2).start()

        @pl.loop(0, _ITEMS_PW, step=_RING)
        def _(j):
            for b in range(_RING):
                il = j + b
                m = b % 2
                gather_desc(il, b).wait()
                if b < 2:
                    @pl.when(j > 0)
                    def _():
                        out_desc(il - 2, m).wait()
                else:
                    out_desc(il - 2, m).wait()
                compute(il, b, m)
                out_desc(il, m).start()

                q3 = j + b + 3

                @pl.when(q3 < _ITEMS_PW)
                def _():
                    gather_desc(il + 3, (b + 3) % _RING).start()

        out_desc(_ITEMS_PW - 2, 0).wait()
        out_desc(_ITEMS_PW - 1, 1).wait()

    return k(xt1, table_p, pos)


def kernel(x, table):
    xt1 = x.astype(jnp.int32).T.reshape(-1)
    table_p = jnp.pad(table, ((0, 0), (0, _PAD - _DIM)))
    out3 = _embed_sc(xt1, table_p, jnp.asarray(_POS))
    return out3.transpose(2, 0, 1)


# speedup vs baseline: 1.6370x; 1.2373x over previous
import functools

import numpy as np
import jax
from jax import lax
import jax.numpy as jnp
from jax.experimental import pallas as pl
from jax.experimental.pallas import tpu as pltpu
from jax.experimental.pallas import tpu_sc as plsc

_DIM = 64
_PAD = 128
_SEQ = 200
_BATCH = 4096
_LANES = 16
_NW = 32
_SPW = _BATCH // _NW
_G0 = 104
_G1 = _SEQ - _G0
_ITEMS = _SPW * 2
_RING = 4


def _positional_encoding(length, dim):
    depth = dim / 2
    positions = np.arange(length)[:, np.newaxis]
    depths = np.arange(int(depth))[np.newaxis, :] / depth
    angle_rates = 1 / 10000 ** depths
    angle_rads = positions * angle_rates
    return np.concatenate(
        [np.sin(angle_rads), np.cos(angle_rads)], axis=-1
    ).astype(np.float32)


_POS = _positional_encoding(_SEQ, _DIM).reshape(-1)


@jax.jit
def _embed_sc(x1, table_p, pos):
    mesh = plsc.VectorSubcoreMesh(
        core_axis_name="core", subcore_axis_name="subcore"
    )

    @functools.partial(
        pl.kernel,
        out_type=jax.ShapeDtypeStruct((_BATCH * _SEQ, _DIM), jnp.float32),
        mesh=mesh,
        scratch_types=[
            pltpu.VMEM((_SEQ * _DIM,), jnp.float32),
            pltpu.VMEM((_SPW * _SEQ,), jnp.int32),
            [pltpu.VMEM((_G0, _PAD), jnp.float32) for _ in range(_RING)],
            [pltpu.VMEM((_G0, _DIM), jnp.float32) for _ in range(2)],
            [pltpu.SemaphoreType.DMA for _ in range(_RING)],
            [pltpu.SemaphoreType.DMA for _ in range(2)],
        ],
        compiler_params=pltpu.CompilerParams(use_tc_tiling_on_sc=True),
    )
    def k(x_hbm, table_hbm, pos_hbm, o_hbm,
          pos_v, idx_v, rows, obufs, gsems, osems):
        cid = lax.axis_index("core")
        sid = lax.axis_index("subcore")
        base = (sid * 2 + cid) * _SPW

        pltpu.sync_copy(pos_hbm, pos_v)
        pltpu.sync_copy(x_hbm.at[pl.ds(base * _SEQ, _SPW * _SEQ)], idx_v)

        def gather_desc(seq, half, b):
            n = _G0 if half == 0 else _G1
            i0 = seq * _SEQ + half * _G0
            return pltpu.make_async_copy(
                table_hbm.at[idx_v.at[pl.ds(i0, n)]],
                rows[b].at[pl.ds(0, n)],
                gsems[b],
            )

        def out_desc(seq, half):
            n = _G0 if half == 0 else _G1
            row0 = (base + seq) * _SEQ + half * _G0
            return pltpu.make_async_copy(
                obufs[half].at[pl.ds(0, n)],
                o_hbm.at[pl.ds(row0, n)],
                osems[half],
            )

        def compute(half, b):
            n = _G0 if half == 0 else _G1
            t0 = half * _G0
            src = rows[b]
            dst = obufs[half]

            @plsc.parallel_loop(0, n, step=2, unroll=4)
            def _(r):
                for rr in range(2):
                    row_s = src.at[r + rr]
                    row_d = dst.at[r + rr]
                    p0 = (t0 + r + rr) * _DIM
                    for c in range(0, _DIM, _LANES):
                        row_d.at[pl.ds(c, _LANES)][...] = (
                            row_s.at[pl.ds(c, _LANES)][...] * 8.0
                            + pos_v.at[pl.ds(p0 + c, _LANES)][...]
                        )

        gather_desc(0, 0, 0).start()
        gather_desc(0, 1, 1).start()
        gather_desc(1, 0, 2).start()

        @pl.loop(0, _ITEMS, step=_RING)
        def _(j):
            seq0 = j // 2
            for b in range(_RING):
                half = b % 2
                seq = seq0 + b // 2
                gather_desc(seq, half, b).wait()
                if b < 2:
                    @pl.when(j > 0)
                    def _():
                        out_desc(seq - 1, half).wait()
                else:
                    out_desc(seq - 1, half).wait()
                compute(half, b)
                out_desc(seq, half).start()

                q3 = j + b + 3

                @pl.when(q3 < _ITEMS)
                def _():
                    gather_desc(
                        j // 2 + (b + 3) // 2, (b + 3) % 2, (b + 3) % _RING
                    ).start()

        out_desc(_SPW - 1, 0).wait()
        out_desc(_SPW - 1, 1).wait()

    return k(x1, table_p, pos)


def kernel(x, table):
    x1 = x.astype(jnp.int32).reshape(-1)
    table_p = jnp.pad(table, ((0, 0), (0, _PAD - _DIM)))
    out = _embed_sc(x1, table_p, jnp.asarray(_POS))
    return out.reshape(_BATCH, _SEQ, _DIM)


# speedup vs baseline: 1.6385x; 1.0009x over previous
import functools

import numpy as np
import jax
from jax import lax
import jax.numpy as jnp
from jax.experimental import pallas as pl
from jax.experimental.pallas import tpu as pltpu
from jax.experimental.pallas import tpu_sc as plsc

_DIM = 64
_PAD = 128
_SEQ = 200
_BATCH = 4096
_LANES = 16
_NW = 32
_SPW = _BATCH // _NW
_G0 = 104
_G1 = _SEQ - _G0
_ITEMS = _SPW * 2
_RING = 4


def _positional_encoding(length, dim):
    depth = dim / 2
    positions = np.arange(length)[:, np.newaxis]
    depths = np.arange(int(depth))[np.newaxis, :] / depth
    angle_rates = 1 / 10000 ** depths
    angle_rads = positions * angle_rates
    return np.concatenate(
        [np.sin(angle_rads), np.cos(angle_rads)], axis=-1
    ).astype(np.float32)


_POS = _positional_encoding(_SEQ, _DIM).reshape(-1)


@jax.jit
def _embed_sc(x1, table_p, pos):
    mesh = plsc.VectorSubcoreMesh(
        core_axis_name="core", subcore_axis_name="subcore"
    )

    @functools.partial(
        pl.kernel,
        out_type=jax.ShapeDtypeStruct((_BATCH * _SEQ, _DIM), jnp.float32),
        mesh=mesh,
        scratch_types=[
            pltpu.VMEM((_SEQ * _DIM,), jnp.float32),
            pltpu.VMEM((_SPW * _SEQ,), jnp.int32),
            [pltpu.VMEM((_G0, _PAD), jnp.float32) for _ in range(_RING)],
            [pltpu.VMEM((_G0, _DIM), jnp.float32) for _ in range(2)],
            [pltpu.SemaphoreType.DMA for _ in range(_RING)],
            [pltpu.SemaphoreType.DMA for _ in range(2)],
        ],
        compiler_params=pltpu.CompilerParams(use_tc_tiling_on_sc=True),
    )
    def k(x_hbm, table_hbm, pos_hbm, o_hbm,
          pos_v, idx_v, rows, obufs, gsems, osems):
        cid = lax.axis_index("core")
        sid = lax.axis_index("subcore")
        base = (sid * 2 + cid) * _SPW

        pltpu.sync_copy(pos_hbm, pos_v)
        pltpu.sync_copy(x_hbm.at[pl.ds(base * _SEQ, _SPW * _SEQ)], idx_v)

        def gather_desc(seq, half, b):
            n = _G0 if half == 0 else _G1
            i0 = seq * _SEQ + half * _G0
            return pltpu.make_async_copy(
                table_hbm.at[idx_v.at[pl.ds(i0, n)]],
                rows[b].at[pl.ds(0, n)],
                gsems[b],
            )

        def out_desc(seq, half):
            n = _G0 if half == 0 else _G1
            row0 = (base + seq) * _SEQ + half * _G0
            return pltpu.make_async_copy(
                obufs[half].at[pl.ds(0, n)],
                o_hbm.at[pl.ds(row0, n)],
                osems[half],
            )

        def compute(half, b):
            n = _G0 if half == 0 else _G1
            t0 = half * _G0
            src = rows[b]
            dst = obufs[half]

            @plsc.parallel_loop(0, n, step=2, unroll=8)
            def _(r):
                for rr in range(2):
                    row_s = src.at[r + rr]
                    row_d = dst.at[r + rr]
                    p0 = (t0 + r + rr) * _DIM
                    for c in range(0, _DIM, _LANES):
                        row_d.at[pl.ds(c, _LANES)][...] = (
                            row_s.at[pl.ds(c, _LANES)][...] * 8.0
                            + pos_v.at[pl.ds(p0 + c, _LANES)][...]
                        )

        gather_desc(0, 0, 0).start()
        gather_desc(0, 1, 1).start()
        gather_desc(1, 0, 2).start()

        @pl.loop(0, _ITEMS, step=_RING)
        def _(j):
            seq0 = j // 2
            for b in range(_RING):
                half = b % 2
                seq = seq0 + b // 2
                gather_desc(seq, half, b).wait()
                if b < 2:
                    @pl.when(j > 0)
                    def _():
                        out_desc(seq - 1, half).wait()
                else:
                    out_desc(seq - 1, half).wait()
                compute(half, b)
                out_desc(seq, half).start()

                q3 = j + b + 3

                @pl.when(q3 < _ITEMS)
                def _():
                    gather_desc(
                        j // 2 + (b + 3) // 2, (b + 3) % 2, (b + 3) % _RING
                    ).start()

        out_desc(_SPW - 1, 0).wait()
        out_desc(_SPW - 1, 1).wait()

    return k(x1, table_p, pos)


def kernel(x, table):
    x1 = x.astype(jnp.int32).reshape(-1)
    table_p = jnp.pad(table, ((0, 0), (0, _PAD - _DIM)))
    out = _embed_sc(x1, table_p, jnp.asarray(_POS))
    return out.reshape(_BATCH, _SEQ, _DIM)
